# baseline (speedup 1.0000x reference)
import functools

import jax
import jax.numpy as jnp
from jax.experimental import pallas as pl
from jax.experimental.pallas import tpu as pltpu

_N_BLK = 1000
_BP = 128
_MIN_SCORE = 0.8
_NEG_INF = float("-inf")


def _body(x_ref, batch_ref, pool_w_ref, aW1_ref, ab1_ref, aW2_ref, ab2_ref,
          fW1_ref, fb1_ref, fW2_ref, fb2_ref,
          out_ref,
          m_ref, s_ref, gm_ref, gs_ref, pooled_ref, x_vmem, raw_vmem, *, nb):
    p = pl.program_id(0)
    i = pl.program_id(1)
    c = x_ref.shape[1]
    rows = pl.ds(i * _N_BLK, _N_BLK)
    dotn = functools.partial(jax.lax.dot_general,
                             preferred_element_type=jnp.float32)

    @pl.when(jnp.logical_and(p == 0, i == 0))
    def _init():
        m_ref[...] = jnp.full((1, _BP), _NEG_INF, jnp.float32)
        s_ref[...] = jnp.zeros((1, _BP), jnp.float32)
        gm_ref[...] = jnp.full((1, _BP), _NEG_INF, jnp.float32)
        gs_ref[...] = jnp.zeros((1, _BP), jnp.float32)
        pooled_ref[...] = jnp.zeros((_BP, c), jnp.float32)

    bid = batch_ref[...]
    onehot = bid == jax.lax.broadcasted_iota(jnp.int32, (_N_BLK, _BP), 1)

    @pl.when(p == 0)
    def _phase0():
        xb = x_ref[...]
        x_vmem[rows, :] = xb
        raw = dotn(xb, pool_w_ref[...], (((1,), (1,)), ((), ())))
        raw_vmem[rows, :] = raw
        d = jnp.where(onehot, raw, _NEG_INF)
        blk_m = jnp.max(d, axis=0, keepdims=True)
        m_old = m_ref[...]
        m_new = jnp.maximum(m_old, blk_m)
        factor = jnp.where(m_new == m_old, 1.0, jnp.exp(m_old - m_new))
        m_sub = jnp.where(m_new == _NEG_INF, 0.0, m_new)
        blk_s = jnp.sum(jnp.exp(d - m_sub), axis=0, keepdims=True)
        s_ref[...] = s_ref[...] * factor + blk_s
        m_ref[...] = m_new

    @pl.when(p == 1)
    def _phase1():
        xb = x_vmem[rows, :]
        raw = raw_vmem[rows, :]
        m_row = m_ref[...]
        s_row = s_ref[...]
        m_sub = jnp.where(m_row == _NEG_INF, 0.0, m_row)
        r_row = jnp.where(s_row > 0, 1.0 / s_row, 0.0)
        smin_row = jnp.where(s_row > 0,
                             jnp.minimum(r_row - 1e-7, _MIN_SCORE),
                             jnp.inf)
        d = jnp.where(onehot, raw, _NEG_INF)
        sc = jnp.exp(d - m_sub) * r_row
        msk = sc > smin_row
        score = jnp.sum(sc, axis=1, keepdims=True)

        y = dotn(xb, aW1_ref[...], (((1,), (0,)), ((), ())))
        pre = score * y + ab1_ref[...]
        g1 = jnp.where(pre >= 0, pre, 0.01 * pre)
        gate = dotn(g1, aW2_ref[...], (((1,), (0,)), ((), ()))) + ab2_ref[...]

        dg = jnp.where(msk, gate, _NEG_INF)
        blk_gm = jnp.max(dg, axis=0, keepdims=True)
        gm_old = gm_ref[...]
        gm_new = jnp.maximum(gm_old, blk_gm)
        factor = jnp.where(gm_new == gm_old, 1.0, jnp.exp(gm_old - gm_new))
        gm_sub = jnp.where(gm_new == _NEG_INF, 0.0, gm_new)
        eg = jnp.exp(dg - gm_sub)
        blk_gs = jnp.sum(eg, axis=0, keepdims=True)
        contrib = dotn(eg, score * xb, (((0,), (0,)), ((), ())))
        gs_ref[...] = gs_ref[...] * factor + blk_gs
        gm_ref[...] = gm_new
        pooled_ref[...] = pooled_ref[...] * factor.T + contrib

        @pl.when(i == nb - 1)
        def _final():
            gs = gs_ref[...]
            inv = jnp.where(gs > 0, 1.0 / gs, 0.0)
            pooled = pooled_ref[...] * inv.T
            h_pre = dotn(pooled, fW1_ref[...],
                         (((1,), (0,)), ((), ()))) + fb1_ref[...]
            h = jnp.where(h_pre >= 0, h_pre, 0.01 * h_pre)
            out_ref[...] = dotn(h, fW2_ref[...],
                                (((1,), (0,)), ((), ()))) + fb2_ref[...]


def _run(x_p, batch2d, pool_w, aW1, ab1r, aW2, ab2r, fW1, fb1r, fW2, fb2r, *, nb, c):
    const = lambda p, i: (0, 0)
    return pl.pallas_call(
        functools.partial(_body, nb=nb),
        grid=(2, nb),
        in_specs=[
            pl.BlockSpec((_N_BLK, c),
                         lambda p, i: (jnp.where(p == 0, i, nb - 1), 0)),
            pl.BlockSpec((_N_BLK, 1), lambda p, i: (i, 0)),
            pl.BlockSpec((1, c), const),
            pl.BlockSpec((c, c), const),
            pl.BlockSpec((1, c), const),
            pl.BlockSpec((c, 1), const),
            pl.BlockSpec((1, 1), const),
            pl.BlockSpec((c, c), const),
            pl.BlockSpec((1, c), const),
            pl.BlockSpec((c, 1), const),
            pl.BlockSpec((1, 1), const),
        ],
        out_specs=pl.BlockSpec((_BP, 1), const),
        out_shape=jax.ShapeDtypeStruct((_BP, 1), jnp.float32),
        scratch_shapes=[
            pltpu.VMEM((1, _BP), jnp.float32),
            pltpu.VMEM((1, _BP), jnp.float32),
            pltpu.VMEM((1, _BP), jnp.float32),
            pltpu.VMEM((1, _BP), jnp.float32),
            pltpu.VMEM((_BP, c), jnp.float32),
            pltpu.VMEM((nb * _N_BLK, c), jnp.float32),
            pltpu.VMEM((nb * _N_BLK, 1), jnp.float32),
        ],
        compiler_params=pltpu.CompilerParams(
            dimension_semantics=("arbitrary", "arbitrary")),
    )(x_p, batch2d, pool_w, aW1, ab1r, aW2, ab2r, fW1, fb1r, fW2, fb2r)


def kernel(x, edge_index, batch, final_output, pool_w, aW1, ab1, aW2, ab2,
           fW1, fb1, fW2, fb2):
    n, c = x.shape
    b = final_output.shape[0]
    nb = pl.cdiv(n, _N_BLK)
    n_pad = nb * _N_BLK
    if n_pad == n:
        x_p = x
        batch_p = batch.astype(jnp.int32)
    else:
        x_p = jnp.pad(x, ((0, n_pad - n), (0, 0)))
        batch_p = jnp.pad(batch.astype(jnp.int32), (0, n_pad - n),
                          constant_values=b)
    batch2d = batch_p.reshape(n_pad, 1)
    out = _run(x_p, batch2d, pool_w, aW1, ab1.reshape(1, c), aW2,
               ab2.reshape(1, 1), fW1, fb1.reshape(1, c), fW2,
               fb2.reshape(1, 1), nb=nb, c=c)
    return jnp.concatenate([final_output, out[:b]], axis=-1)


# speedup vs baseline: 1.2609x; 1.2609x over previous
import functools

import jax
import jax.numpy as jnp
from jax.experimental import pallas as pl
from jax.experimental.pallas import tpu as pltpu

_N_BLK = 5000
_BP = 128
_MIN_SCORE = 0.8
_NEG_INF = float("-inf")


def _body(x_ref, batch_ref, pool_w_ref, aW1_ref, ab1_ref, aW2_ref, ab2_ref,
          fW1_ref, fb1_ref, fW2_ref, fb2_ref,
          out_ref,
          m_ref, s_ref, gm_ref, gs_ref, pooled_ref, x_vmem, raw_vmem, *, nb):
    p = pl.program_id(0)
    i = pl.program_id(1)
    c = x_ref.shape[1]
    rows = pl.ds(i * _N_BLK, _N_BLK)
    dotn = functools.partial(jax.lax.dot_general,
                             preferred_element_type=jnp.float32)

    @pl.when(jnp.logical_and(p == 0, i == 0))
    def _init():
        m_ref[...] = jnp.full((1, _BP), _NEG_INF, jnp.float32)
        s_ref[...] = jnp.zeros((1, _BP), jnp.float32)
        gm_ref[...] = jnp.full((1, _BP), _NEG_INF, jnp.float32)
        gs_ref[...] = jnp.zeros((1, _BP), jnp.float32)
        pooled_ref[...] = jnp.zeros((_BP, c), jnp.float32)

    bid = batch_ref[...]
    onehot = bid == jax.lax.broadcasted_iota(jnp.int32, (_N_BLK, _BP), 1)

    @pl.when(p == 0)
    def _phase0():
        xb = x_ref[...]
        x_vmem[rows, :] = xb
        raw = dotn(xb, pool_w_ref[...], (((1,), (1,)), ((), ())))
        raw_vmem[rows, :] = raw
        d = jnp.where(onehot, raw, _NEG_INF)
        blk_m = jnp.max(d, axis=0, keepdims=True)
        m_old = m_ref[...]
        m_new = jnp.maximum(m_old, blk_m)
        factor = jnp.where(m_new == m_old, 1.0, jnp.exp(m_old - m_new))
        m_sub = jnp.where(m_new == _NEG_INF, 0.0, m_new)
        blk_s = jnp.sum(jnp.exp(d - m_sub), axis=0, keepdims=True)
        s_ref[...] = s_ref[...] * factor + blk_s
        m_ref[...] = m_new

    @pl.when(p == 1)
    def _phase1():
        xb = x_vmem[rows, :]
        raw = raw_vmem[rows, :]
        m_row = m_ref[...]
        s_row = s_ref[...]
        m_sub = jnp.where(m_row == _NEG_INF, 0.0, m_row)
        r_row = jnp.where(s_row > 0, 1.0 / s_row, 0.0)
        smin_row = jnp.where(s_row > 0,
                             jnp.minimum(r_row - 1e-7, _MIN_SCORE),
                             jnp.inf)
        d = jnp.where(onehot, raw, _NEG_INF)
        sc = jnp.exp(d - m_sub) * r_row
        msk = sc > smin_row
        score = jnp.sum(sc, axis=1, keepdims=True)

        y = dotn(xb, aW1_ref[...], (((1,), (0,)), ((), ())))
        pre = score * y + ab1_ref[...]
        g1 = jnp.where(pre >= 0, pre, 0.01 * pre)
        gate = dotn(g1, aW2_ref[...], (((1,), (0,)), ((), ()))) + ab2_ref[...]

        dg = jnp.where(msk, gate, _NEG_INF)
        blk_gm = jnp.max(dg, axis=0, keepdims=True)
        gm_old = gm_ref[...]
        gm_new = jnp.maximum(gm_old, blk_gm)
        factor = jnp.where(gm_new == gm_old, 1.0, jnp.exp(gm_old - gm_new))
        gm_sub = jnp.where(gm_new == _NEG_INF, 0.0, gm_new)
        eg = jnp.exp(dg - gm_sub)
        blk_gs = jnp.sum(eg, axis=0, keepdims=True)
        contrib = dotn(eg, score * xb, (((0,), (0,)), ((), ())))
        gs_ref[...] = gs_ref[...] * factor + blk_gs
        gm_ref[...] = gm_new
        pooled_ref[...] = pooled_ref[...] * factor.T + contrib

        @pl.when(i == nb - 1)
        def _final():
            gs = gs_ref[...]
            inv = jnp.where(gs > 0, 1.0 / gs, 0.0)
            pooled = pooled_ref[...] * inv.T
            h_pre = dotn(pooled, fW1_ref[...],
                         (((1,), (0,)), ((), ()))) + fb1_ref[...]
            h = jnp.where(h_pre >= 0, h_pre, 0.01 * h_pre)
            out_ref[...] = dotn(h, fW2_ref[...],
                                (((1,), (0,)), ((), ()))) + fb2_ref[...]


def _run(x_p, batch2d, pool_w, aW1, ab1r, aW2, ab2r, fW1, fb1r, fW2, fb2r, *, nb, c):
    const = lambda p, i: (0, 0)
    return pl.pallas_call(
        functools.partial(_body, nb=nb),
        grid=(2, nb),
        in_specs=[
            pl.BlockSpec((_N_BLK, c),
                         lambda p, i: (jnp.where(p == 0, i, nb - 1), 0)),
            pl.BlockSpec((_N_BLK, 1), lambda p, i: (i, 0)),
            pl.BlockSpec((1, c), const),
            pl.BlockSpec((c, c), const),
            pl.BlockSpec((1, c), const),
            pl.BlockSpec((c, 1), const),
            pl.BlockSpec((1, 1), const),
            pl.BlockSpec((c, c), const),
            pl.BlockSpec((1, c), const),
            pl.BlockSpec((c, 1), const),
            pl.BlockSpec((1, 1), const),
        ],
        out_specs=pl.BlockSpec((_BP, 1), const),
        out_shape=jax.ShapeDtypeStruct((_BP, 1), jnp.float32),
        scratch_shapes=[
            pltpu.VMEM((1, _BP), jnp.float32),
            pltpu.VMEM((1, _BP), jnp.float32),
            pltpu.VMEM((1, _BP), jnp.float32),
            pltpu.VMEM((1, _BP), jnp.float32),
            pltpu.VMEM((_BP, c), jnp.float32),
            pltpu.VMEM((nb * _N_BLK, c), jnp.float32),
            pltpu.VMEM((nb * _N_BLK, 1), jnp.float32),
        ],
        compiler_params=pltpu.CompilerParams(
            dimension_semantics=("arbitrary", "arbitrary")),
    )(x_p, batch2d, pool_w, aW1, ab1r, aW2, ab2r, fW1, fb1r, fW2, fb2r)


def kernel(x, edge_index, batch, final_output, pool_w, aW1, ab1, aW2, ab2,
           fW1, fb1, fW2, fb2):
    n, c = x.shape
    b = final_output.shape[0]
    nb = pl.cdiv(n, _N_BLK)
    n_pad = nb * _N_BLK
    if n_pad == n:
        x_p = x
        batch_p = batch.astype(jnp.int32)
    else:
        x_p = jnp.pad(x, ((0, n_pad - n), (0, 0)))
        batch_p = jnp.pad(batch.astype(jnp.int32), (0, n_pad - n),
                          constant_values=b)
    batch2d = batch_p.reshape(n_pad, 1)
    out = _run(x_p, batch2d, pool_w, aW1, ab1.reshape(1, c), aW2,
               ab2.reshape(1, 1), fW1, fb1.reshape(1, c), fW2,
               fb2.reshape(1, 1), nb=nb, c=c)
    return jnp.concatenate([final_output, out[:b]], axis=-1)
